# baseline (speedup 1.0000x reference)
import functools

import jax
import jax.numpy as jnp
from jax import lax
from jax.experimental import pallas as pl
from jax.experimental.pallas import tpu as pltpu
from jax.experimental.pallas import tpu_sc as plsc

_NUM_CORES = 2
_NUM_SUBCORES = 16
_LANES = 16
_NW = _NUM_CORES * _NUM_SUBCORES

_GATHER_CHUNK = 128


def _sc_body(n_nodes, n_edges, b_pad, f_dim,
             x_hbm, atom_hbm, seq_hbm, partials_hbm, g_hbm,
             idx_v, rows_v, src_v, dst_v, hist_v, gsem, hsem, esem):
    ew = n_edges // _NW
    bw = b_pad // _NW
    wid = lax.axis_index("s") * _NUM_CORES + lax.axis_index("c")

    gbase = wid * bw
    with jax.named_scope("ph1_gather_issue"):
        pltpu.sync_copy(atom_hbm.at[pl.ds(gbase, bw)], idx_v)
        n_chunks = bw // _GATHER_CHUNK

    ebase = wid * ew
    with jax.named_scope("ph2_edge_issue"):
        pltpu.async_copy(seq_hbm.at[pl.ds(ebase, ew)], src_v, esem)
        pltpu.async_copy(seq_hbm.at[pl.ds(n_edges + ebase, ew)], dst_v, esem)

    zeros = jnp.zeros((_LANES,), jnp.int32)

    with jax.named_scope("ph3_zero"):
        @plsc.parallel_loop(0, 3 * n_nodes, step=_LANES, unroll=8)
        def _(i):
            hist_v[pl.ds(i, _LANES)] = zeros

    with jax.named_scope("ph4_edge_wait"):
        pltpu.make_async_copy(seq_hbm.at[pl.ds(ebase, ew)], src_v, esem).wait()
        pltpu.make_async_copy(seq_hbm.at[pl.ds(n_edges + ebase, ew)], dst_v,
                              esem).wait()

    with jax.named_scope("ph5_hist"):
        @plsc.parallel_loop(0, ew, step=_LANES, unroll=4)
        def _(i):
            s = src_v[pl.ds(i, _LANES)]
            d = dst_v[pl.ds(i, _LANES)]
            bins = (d - s + 1) * n_nodes + s
            cnt, last = plsc.scan_count(bins)
            plsc.addupdate_scatter(hist_v, [bins], cnt, mask=last)

    with jax.named_scope("ph6_hist_dma"):
        for k in range(3):
            pltpu.async_copy(hist_v.at[pl.ds(k * n_nodes, n_nodes)],
                             partials_hbm.at[pl.ds((k * _NW + wid) * n_nodes,
                                                   n_nodes)], hsem)

    with jax.named_scope("ph7_gather_drain"):
        for ci in range(n_chunks):
            sl = pl.ds(ci * _GATHER_CHUNK, _GATHER_CHUNK)
            pltpu.sync_copy(x_hbm.at[idx_v.at[sl]], rows_v.at[sl])
    with jax.named_scope("ph8_g_write"):
        pltpu.sync_copy(rows_v, g_hbm.at[pl.ds(gbase, bw)])

    with jax.named_scope("ph9_hist_dma_wait"):
        for k in range(3):
            pltpu.make_async_copy(hist_v.at[pl.ds(k * n_nodes, n_nodes)],
                                  partials_hbm.at[pl.ds((k * _NW + wid) * n_nodes,
                                                        n_nodes)], hsem).wait()


def _tc_body(n_nodes, partials_ref, g_ref, w_ref, out_ref):
    counts = jnp.sum(partials_ref[...], axis=1).astype(jnp.float32)
    coef = lax.dot_general(
        counts, w_ref[...],
        dimension_numbers=(((0,), (0,)), ((), ())),
        preferred_element_type=jnp.float32,
    )
    coef = coef + w_ref[1, :][None, :]
    out_ref[...] = coef * g_ref[0:n_nodes, :]


def kernel(x, atom_types, seq_neighs, weight):
    n_nodes, f_dim = x.shape
    n_edges = seq_neighs.shape[1]
    b_pad = ((n_nodes + _NW * _GATHER_CHUNK - 1)
             // (_NW * _GATHER_CHUNK)) * (_NW * _GATHER_CHUNK)
    atom_pad = jnp.concatenate(
        [atom_types.astype(jnp.int32),
         jnp.zeros((b_pad - n_nodes,), jnp.int32)])

    mesh = plsc.VectorSubcoreMesh(core_axis_name="c", subcore_axis_name="s")
    bw = b_pad // _NW
    sc = pl.kernel(
        functools.partial(_sc_body, n_nodes, n_edges, b_pad, f_dim),
        out_type=(
            jax.ShapeDtypeStruct((3 * _NW * n_nodes,), jnp.int32),
            jax.ShapeDtypeStruct((b_pad, f_dim), jnp.float32),
        ),
        mesh=mesh,
        scratch_types=[
            pltpu.VMEM((bw,), jnp.int32),
            pltpu.VMEM((bw, f_dim), jnp.float32),
            pltpu.VMEM((n_edges // _NW,), jnp.int32),
            pltpu.VMEM((n_edges // _NW,), jnp.int32),
            pltpu.VMEM((3 * n_nodes,), jnp.int32),
            pltpu.SemaphoreType.DMA,
            pltpu.SemaphoreType.DMA,
            pltpu.SemaphoreType.DMA,
        ],
        compiler_params=pltpu.CompilerParams(needs_layout_passes=False),
    )
    partials, g = sc(x, atom_pad, seq_neighs.astype(jnp.int32).reshape(-1))
    partials = partials.reshape(3, _NW, n_nodes)

    out = pl.pallas_call(
        functools.partial(_tc_body, n_nodes),
        out_shape=jax.ShapeDtypeStruct((n_nodes, f_dim), jnp.float32),
    )(partials, g, weight)
    return out


# speedup vs baseline: 3.2764x; 3.2764x over previous
import functools

import jax
import jax.numpy as jnp
from jax import lax
from jax.experimental import pallas as pl
from jax.experimental.pallas import tpu as pltpu
from jax.experimental.pallas import tpu_sc as plsc

_NUM_CORES = 2
_NUM_SUBCORES = 16
_LANES = 16
_NW = _NUM_CORES * _NUM_SUBCORES

_GATHER_CHUNK = 128


def _chunks(total):
    out = []
    off = 0
    while off < total:
        size = min(_GATHER_CHUNK, total - off)
        out.append((off, size))
        off += size
    return out


def _sc_body(n_nodes, n_edges, bw, bw_last, f_dim,
             x_hbm, atom_hbm, seq_hbm, partials_hbm, g_hbm,
             idx_v, rows_v, src_v, dst_v, hist_v, gsem, hsem, esem):
    ew = n_edges // _NW
    wid = lax.axis_index("s") * _NUM_CORES + lax.axis_index("c")
    is_last = wid == _NW - 1

    gbase = wid * bw

    @pl.when(jnp.logical_not(is_last))
    def _():
        pltpu.sync_copy(atom_hbm.at[pl.ds(gbase, bw)], idx_v.at[pl.ds(0, bw)])
        for off, size in _chunks(bw):
            pltpu.async_copy(x_hbm.at[idx_v.at[pl.ds(off, size)]],
                             rows_v.at[pl.ds(off, size)], gsem)

    lbase = (_NW - 1) * bw

    @pl.when(is_last)
    def _():
        pltpu.sync_copy(atom_hbm.at[pl.ds(lbase, bw_last)],
                        idx_v.at[pl.ds(0, bw_last)])
        for off, size in _chunks(bw_last):
            pltpu.async_copy(x_hbm.at[idx_v.at[pl.ds(off, size)]],
                             rows_v.at[pl.ds(off, size)], gsem)

    ebase = wid * ew
    pltpu.async_copy(seq_hbm.at[pl.ds(ebase, ew)], src_v, esem)
    pltpu.async_copy(seq_hbm.at[pl.ds(n_edges + ebase, ew)], dst_v, esem)

    zeros = jnp.zeros((_LANES,), jnp.int32)

    @plsc.parallel_loop(0, 3 * n_nodes, step=_LANES, unroll=8)
    def _(i):
        hist_v[pl.ds(i, _LANES)] = zeros

    pltpu.make_async_copy(seq_hbm.at[pl.ds(ebase, ew)], src_v, esem).wait()
    pltpu.make_async_copy(seq_hbm.at[pl.ds(n_edges + ebase, ew)], dst_v,
                          esem).wait()

    @plsc.parallel_loop(0, ew, step=_LANES, unroll=4)
    def _(i):
        s = src_v[pl.ds(i, _LANES)]
        d = dst_v[pl.ds(i, _LANES)]
        bins = (d - s + 1) * n_nodes + s
        cnt, last = plsc.scan_count(bins)
        plsc.addupdate_scatter(hist_v, [bins], cnt, mask=last)

    for k in range(3):
        pltpu.async_copy(hist_v.at[pl.ds(k * n_nodes, n_nodes)],
                         partials_hbm.at[pl.ds((k * _NW + wid) * n_nodes,
                                               n_nodes)], hsem)

    @pl.when(jnp.logical_not(is_last))
    def _():
        for off, size in _chunks(bw):
            pltpu.make_async_copy(x_hbm.at[idx_v.at[pl.ds(off, size)]],
                                  rows_v.at[pl.ds(off, size)], gsem).wait()
        pltpu.sync_copy(rows_v.at[pl.ds(0, bw)], g_hbm.at[pl.ds(gbase, bw)])

    @pl.when(is_last)
    def _():
        for off, size in _chunks(bw_last):
            pltpu.make_async_copy(x_hbm.at[idx_v.at[pl.ds(off, size)]],
                                  rows_v.at[pl.ds(off, size)], gsem).wait()
        pltpu.sync_copy(rows_v.at[pl.ds(0, bw_last)],
                        g_hbm.at[pl.ds(lbase, bw_last)])

    for k in range(3):
        pltpu.make_async_copy(hist_v.at[pl.ds(k * n_nodes, n_nodes)],
                              partials_hbm.at[pl.ds((k * _NW + wid) * n_nodes,
                                                    n_nodes)], hsem).wait()


def _tc_body(partials_ref, g_ref, w_ref, out_ref):
    counts = jnp.sum(partials_ref[...], axis=1).astype(jnp.float32)
    coef = lax.dot_general(
        counts, w_ref[...],
        dimension_numbers=(((0,), (0,)), ((), ())),
        preferred_element_type=jnp.float32,
    )
    coef = coef + w_ref[1, :][None, :]
    out_ref[...] = coef * g_ref[...]


def kernel(x, atom_types, seq_neighs, weight):
    n_nodes, f_dim = x.shape
    n_edges = seq_neighs.shape[1]
    bw = (-(-n_nodes // _NW) + 7) // 8 * 8
    bw_last = n_nodes - bw * (_NW - 1)
    assert 0 < bw_last <= bw

    mesh = plsc.VectorSubcoreMesh(core_axis_name="c", subcore_axis_name="s")
    sc = pl.kernel(
        functools.partial(_sc_body, n_nodes, n_edges, bw, bw_last, f_dim),
        out_type=(
            jax.ShapeDtypeStruct((3 * _NW * n_nodes,), jnp.int32),
            jax.ShapeDtypeStruct((n_nodes, f_dim), jnp.float32),
        ),
        mesh=mesh,
        scratch_types=[
            pltpu.VMEM((bw,), jnp.int32),
            pltpu.VMEM((bw, f_dim), jnp.float32),
            pltpu.VMEM((n_edges // _NW,), jnp.int32),
            pltpu.VMEM((n_edges // _NW,), jnp.int32),
            pltpu.VMEM((3 * n_nodes,), jnp.int32),
            pltpu.SemaphoreType.DMA,
            pltpu.SemaphoreType.DMA,
            pltpu.SemaphoreType.DMA,
        ],
        compiler_params=pltpu.CompilerParams(needs_layout_passes=False),
    )
    partials, g = sc(x, atom_types.astype(jnp.int32),
                     seq_neighs.astype(jnp.int32).reshape(-1))
    partials = partials.reshape(3, _NW, n_nodes)

    out = pl.pallas_call(
        _tc_body,
        out_shape=jax.ShapeDtypeStruct((n_nodes, f_dim), jnp.float32),
    )(partials, g, weight)
    return out


# speedup vs baseline: 3.6057x; 1.1005x over previous
import functools

import jax
import jax.numpy as jnp
from jax import lax
from jax.experimental import pallas as pl
from jax.experimental.pallas import tpu as pltpu
from jax.experimental.pallas import tpu_sc as plsc

_NUM_CORES = 2
_NUM_SUBCORES = 16
_LANES = 16
_NW = _NUM_CORES * _NUM_SUBCORES

_GATHER_CHUNK = 128


def _chunks(total):
    out = []
    off = 0
    while off < total:
        size = min(_GATHER_CHUNK, total - off)
        out.append((off, size))
        off += size
    return out


_HCOLS = 128
_HROWS_PAD = 256


def _sc_body(n_nodes, n_edges, bw, bw_last, f_dim,
             x_hbm, atom_hbm, seq_hbm, partials_hbm, g_hbm,
             idx_v, rows_v, src_v, dst_v, hist_v, ridx_v, shared_h,
             gsem, hsem, esem):
    ew = n_edges // _NW
    sid = lax.axis_index("s")
    cid = lax.axis_index("c")
    wid = sid * _NUM_CORES + cid
    is_last = wid == _NW - 1
    hrows = 3 * n_nodes // _HCOLS

    gbase = wid * bw

    @pl.when(jnp.logical_not(is_last))
    def _():
        pltpu.sync_copy(atom_hbm.at[pl.ds(gbase, bw)], idx_v.at[pl.ds(0, bw)])
        for off, size in _chunks(bw):
            pltpu.async_copy(x_hbm.at[idx_v.at[pl.ds(off, size)]],
                             rows_v.at[pl.ds(off, size)], gsem)

    lbase = (_NW - 1) * bw

    @pl.when(is_last)
    def _():
        pltpu.sync_copy(atom_hbm.at[pl.ds(lbase, bw_last)],
                        idx_v.at[pl.ds(0, bw_last)])
        for off, size in _chunks(bw_last):
            pltpu.async_copy(x_hbm.at[idx_v.at[pl.ds(off, size)]],
                             rows_v.at[pl.ds(off, size)], gsem)

    ebase = wid * ew
    pltpu.async_copy(seq_hbm.at[pl.ds(ebase, ew)], src_v, esem)
    pltpu.async_copy(seq_hbm.at[pl.ds(n_edges + ebase, ew)], dst_v, esem)

    zeros = jnp.zeros((_LANES,), jnp.int32)

    @plsc.parallel_loop(0, _HROWS_PAD, step=1, unroll=2)
    def _(r):
        for c in range(0, _HCOLS, _LANES):
            hist_v[r, pl.ds(c, _LANES)] = zeros

    my_rows = pl.ds(sid * (_HROWS_PAD // _NUM_SUBCORES),
                    _HROWS_PAD // _NUM_SUBCORES)
    pltpu.sync_copy(hist_v.at[my_rows], shared_h.at[my_rows])

    lane_iota = jnp.arange(_LANES, dtype=jnp.int32)
    for j in range(0, _HROWS_PAD, _LANES):
        ridx_v[j // 128, pl.ds(j % 128, _LANES)] = lane_iota + j

    pltpu.make_async_copy(seq_hbm.at[pl.ds(ebase, ew)], src_v, esem).wait()
    pltpu.make_async_copy(seq_hbm.at[pl.ds(n_edges + ebase, ew)], dst_v,
                          esem).wait()

    @plsc.parallel_loop(0, ew, step=_LANES, unroll=4)
    def _(i):
        s = src_v[pl.ds(i, _LANES)]
        d = dst_v[pl.ds(i, _LANES)]
        bins = (d - s + 1) * n_nodes + s
        rows = bins >> 7
        cols = bins & (_HCOLS - 1)
        cnt, last = plsc.scan_count(bins)
        plsc.addupdate_scatter(hist_v, [rows, cols], cnt, mask=last)

    plsc.subcore_barrier()
    for j in range(_HROWS_PAD // 128):
        pltpu.sync_copy(hist_v.at[pl.ds(j * 128, 128)],
                        shared_h.at[ridx_v.at[j]], add=True)
    plsc.subcore_barrier()
    pltpu.async_copy(
        shared_h.at[my_rows],
        partials_hbm.at[pl.ds(cid * _HROWS_PAD
                              + sid * (_HROWS_PAD // _NUM_SUBCORES),
                              _HROWS_PAD // _NUM_SUBCORES)], hsem)

    @pl.when(jnp.logical_not(is_last))
    def _():
        for off, size in _chunks(bw):
            pltpu.make_async_copy(x_hbm.at[idx_v.at[pl.ds(off, size)]],
                                  rows_v.at[pl.ds(off, size)], gsem).wait()
        pltpu.sync_copy(rows_v.at[pl.ds(0, bw)], g_hbm.at[pl.ds(gbase, bw)])

    @pl.when(is_last)
    def _():
        for off, size in _chunks(bw_last):
            pltpu.make_async_copy(x_hbm.at[idx_v.at[pl.ds(off, size)]],
                                  rows_v.at[pl.ds(off, size)], gsem).wait()
        pltpu.sync_copy(rows_v.at[pl.ds(0, bw_last)],
                        g_hbm.at[pl.ds(lbase, bw_last)])

    pltpu.make_async_copy(
        shared_h.at[my_rows],
        partials_hbm.at[pl.ds(cid * _HROWS_PAD
                              + sid * (_HROWS_PAD // _NUM_SUBCORES),
                              _HROWS_PAD // _NUM_SUBCORES)], hsem).wait()


def _tc_body(partials_ref, g_ref, w_ref, out_ref):
    counts = jnp.sum(partials_ref[...], axis=0).astype(jnp.float32)
    coef = lax.dot_general(
        counts, w_ref[...],
        dimension_numbers=(((0,), (0,)), ((), ())),
        preferred_element_type=jnp.float32,
    )
    coef = coef + w_ref[1, :][None, :]
    out_ref[...] = coef * g_ref[...]


def kernel(x, atom_types, seq_neighs, weight):
    n_nodes, f_dim = x.shape
    n_edges = seq_neighs.shape[1]
    bw = (-(-n_nodes // _NW) + 7) // 8 * 8
    bw_last = n_nodes - bw * (_NW - 1)
    assert 0 < bw_last <= bw

    mesh = plsc.VectorSubcoreMesh(core_axis_name="c", subcore_axis_name="s")
    sc = pl.kernel(
        functools.partial(_sc_body, n_nodes, n_edges, bw, bw_last, f_dim),
        out_type=(
            jax.ShapeDtypeStruct((_NUM_CORES * _HROWS_PAD, _HCOLS),
                                 jnp.int32),
            jax.ShapeDtypeStruct((n_nodes, f_dim), jnp.float32),
        ),
        mesh=mesh,
        scratch_types=[
            pltpu.VMEM((bw,), jnp.int32),
            pltpu.VMEM((bw, f_dim), jnp.float32),
            pltpu.VMEM((n_edges // _NW,), jnp.int32),
            pltpu.VMEM((n_edges // _NW,), jnp.int32),
            pltpu.VMEM((_HROWS_PAD, _HCOLS), jnp.int32),
            pltpu.VMEM((_HROWS_PAD // 128, 128), jnp.int32),
            pltpu.VMEM_SHARED((_HROWS_PAD, _HCOLS), jnp.int32),
            pltpu.SemaphoreType.DMA,
            pltpu.SemaphoreType.DMA,
            pltpu.SemaphoreType.DMA,
        ],
        compiler_params=pltpu.CompilerParams(needs_layout_passes=False),
    )
    partials, g = sc(x, atom_types.astype(jnp.int32),
                     seq_neighs.astype(jnp.int32).reshape(-1))
    partials = (partials.reshape(_NUM_CORES, _HROWS_PAD * _HCOLS)
                [:, :3 * n_nodes].reshape(_NUM_CORES, 3, n_nodes))

    out = pl.pallas_call(
        _tc_body,
        out_shape=jax.ShapeDtypeStruct((n_nodes, f_dim), jnp.float32),
    )(partials, g, weight)
    return out
